# Initial kernel scaffold; baseline (speedup 1.0000x reference)
import jax
import jax.numpy as jnp
from jax import lax
from jax.experimental import pallas as pl
from jax.experimental.pallas import tpu as pltpu
from jax.experimental.pallas import tpu_sc as plsc

N, D, H = 10000, 128, 128
E = 320000
NC, NS, L = 2, 16, 16
NW = NC * NS
EPT = E // NW
CH = 80
NCHUNK = EPT // CH
RPT = N // NS
ZR = 125
BM = 2000

_MESH = plsc.VectorSubcoreMesh(core_axis_name="c", subcore_axis_name="s")


def _deg_body(dst_hbm, out_hbm, dstv, onesv, zbuf, acc):
    c = lax.axis_index("c")
    s = lax.axis_index("s")
    wid = s * NC + c

    def fill(i, _):
        onesv[i] = jnp.ones((L,), jnp.float32)
        zbuf[i] = jnp.zeros((L,), jnp.float32)
        return 0

    lax.fori_loop(0, CH, fill, 0)

    def fillz(i, _):
        zbuf[i] = jnp.zeros((L,), jnp.float32)
        return 0

    lax.fori_loop(CH, ZR, fillz, 0)
    for j in range(RPT // ZR):
        pltpu.sync_copy(zbuf, acc.at[pl.ds(s * RPT + j * ZR, ZR)])
    plsc.subcore_barrier()

    def body(i, _):
        b = pl.multiple_of(wid * EPT + i * CH, 8)
        pltpu.sync_copy(dst_hbm.at[pl.ds(b, CH)], dstv)
        pltpu.sync_copy(onesv, acc.at[dstv], add=True)
        return 0

    lax.fori_loop(0, NCHUNK, body, 0)
    plsc.subcore_barrier()
    pltpu.sync_copy(acc.at[pl.ds(s * RPT, RPT)],
                    out_hbm.at[c, pl.ds(s * RPT, RPT)])


_sc_deg = pl.kernel(
    _deg_body,
    out_type=jax.ShapeDtypeStruct((NC, N, L), jnp.float32),
    mesh=_MESH,
    scratch_types=[
        pltpu.VMEM((CH,), jnp.int32),
        pltpu.VMEM((CH, L), jnp.float32),
        pltpu.VMEM((ZR, L), jnp.float32),
        pltpu.VMEM_SHARED((N, L), jnp.float32),
    ],
)


def _edge_body(g_hbm, src_hbm, dst_hbm, out_hbm, srcv, dstv, rows, zbuf, acc,
               sem):
    c = lax.axis_index("c")
    s = lax.axis_index("s")
    wid = s * NC + c

    def fillz(i, _):
        for jcol in range(H // L):
            zbuf[i, pl.ds(jcol * L, L)] = jnp.zeros((L,), jnp.float32)
        return 0

    lax.fori_loop(0, ZR, fillz, 0)
    for j in range(RPT // ZR):
        pltpu.sync_copy(zbuf, acc.at[pl.ds(s * RPT + j * ZR, ZR)])
    plsc.subcore_barrier()

    def body(i, _):
        b = pl.multiple_of(wid * EPT + i * CH, 8)
        pltpu.sync_copy(src_hbm.at[pl.ds(b, CH)], srcv)
        pltpu.sync_copy(dst_hbm.at[pl.ds(b, CH)], dstv)
        pltpu.async_copy(g_hbm.at[srcv], rows, sem).wait()
        pltpu.sync_copy(rows, acc.at[dstv], add=True)
        return 0

    lax.fori_loop(0, NCHUNK, body, 0)
    plsc.subcore_barrier()
    pltpu.sync_copy(acc.at[pl.ds(s * RPT, RPT)],
                    out_hbm.at[c, pl.ds(s * RPT, RPT)])


_sc_edge = pl.kernel(
    _edge_body,
    out_type=jax.ShapeDtypeStruct((NC, N, H), jnp.float32),
    mesh=_MESH,
    scratch_types=[
        pltpu.VMEM((CH,), jnp.int32),
        pltpu.VMEM((CH,), jnp.int32),
        pltpu.VMEM((CH, H), jnp.float32),
        pltpu.VMEM((ZR, H), jnp.float32),
        pltpu.VMEM_SHARED((N, H), jnp.float32),
        pltpu.SemaphoreType.DMA,
    ],
)


def _tc_l1_body(x_ref, w_ref, dacc_ref, g_ref, dis_ref):
    deg = dacc_ref[0, :, 0:1] + dacc_ref[1, :, 0:1] + 1.0
    dis = lax.rsqrt(deg)
    dis_ref[...] = dis
    h = lax.dot_general(x_ref[...], w_ref[...], (((1,), (1,)), ((), ())),
                        preferred_element_type=jnp.float32)
    g_ref[...] = h * dis


def _tc_l1(x, W1, dacc):
    return pl.pallas_call(
        _tc_l1_body,
        grid=(N // BM,),
        in_specs=[
            pl.BlockSpec((BM, D), lambda i: (i, 0)),
            pl.BlockSpec((H, D), lambda i: (0, 0)),
            pl.BlockSpec((NC, BM, L), lambda i: (0, i, 0)),
        ],
        out_specs=[
            pl.BlockSpec((BM, H), lambda i: (i, 0)),
            pl.BlockSpec((BM, 1), lambda i: (i, 0)),
        ],
        out_shape=[
            jax.ShapeDtypeStruct((N, H), jnp.float32),
            jax.ShapeDtypeStruct((N, 1), jnp.float32),
        ],
    )(x, W1, dacc)


def _tc_mid_body(acc_ref, g_ref, dis_ref, b_ref, w_ref, out_ref):
    dis = dis_ref[...]
    sfull = acc_ref[0] + acc_ref[1] + g_ref[...]
    a = jnp.maximum(sfull * dis + b_ref[...], 0.0)
    out_ref[...] = lax.dot_general(a, w_ref[...], (((1,), (1,)), ((), ())),
                                   preferred_element_type=jnp.float32) * dis


def _tc_mid(acc, g, dis, b, W):
    return pl.pallas_call(
        _tc_mid_body,
        grid=(N // BM,),
        in_specs=[
            pl.BlockSpec((NC, BM, H), lambda i: (0, i, 0)),
            pl.BlockSpec((BM, H), lambda i: (i, 0)),
            pl.BlockSpec((BM, 1), lambda i: (i, 0)),
            pl.BlockSpec((1, H), lambda i: (0, 0)),
            pl.BlockSpec((H, H), lambda i: (0, 0)),
        ],
        out_specs=pl.BlockSpec((BM, H), lambda i: (i, 0)),
        out_shape=jax.ShapeDtypeStruct((N, H), jnp.float32),
    )(acc, g, dis, b, W)


def _tc_fin_body(acc_ref, g_ref, dis_ref, b_ref, wl_ref, bl_ref, out_ref):
    sfull = acc_ref[0] + acc_ref[1] + g_ref[...]
    a = jnp.maximum(sfull * dis_ref[...] + b_ref[...], 0.0)
    out_ref[...] = jnp.sum(a * wl_ref[...], axis=1, keepdims=True) + bl_ref[...]


def _tc_fin(acc, g, dis, b7, Wl, bl):
    return pl.pallas_call(
        _tc_fin_body,
        grid=(N // BM,),
        in_specs=[
            pl.BlockSpec((NC, BM, H), lambda i: (0, i, 0)),
            pl.BlockSpec((BM, H), lambda i: (i, 0)),
            pl.BlockSpec((BM, 1), lambda i: (i, 0)),
            pl.BlockSpec((1, H), lambda i: (0, 0)),
            pl.BlockSpec((1, H), lambda i: (0, 0)),
            pl.BlockSpec((1, 1), lambda i: (0, 0)),
        ],
        out_specs=pl.BlockSpec((BM, 1), lambda i: (i, 0)),
        out_shape=jax.ShapeDtypeStruct((N, 1), jnp.float32),
    )(acc, g, dis, b7, Wl, bl)


def kernel(x, edge_index, W1, b1, W2, b2, W3, b3, W4, b4, W5, b5, W6, b6,
           W7, b7, Wl, bl):
    src = edge_index[0]
    dst = edge_index[1]
    dacc = _sc_deg(dst)
    g, dis = _tc_l1(x, W1, dacc)
    for Wn, bn in ((W2, b1), (W3, b2), (W4, b3), (W5, b4), (W6, b5),
                   (W7, b6)):
        acc = _sc_edge(g, src, dst)
        g = _tc_mid(acc, g, dis, bn.reshape(1, H), Wn)
    acc = _sc_edge(g, src, dst)
    out = _tc_fin(acc, g, dis, b7.reshape(1, H), Wl, bl.reshape(1, 1))
    return out.reshape(N)


# speedup vs baseline: 8.9712x; 8.9712x over previous
import jax
import jax.numpy as jnp
from jax import lax
from jax.experimental import pallas as pl
from jax.experimental.pallas import tpu as pltpu
from jax.experimental.pallas import tpu_sc as plsc

N, D, H = 10000, 128, 128
E = 320000
NP = 10240
NC, NS, L = 2, 16, 16
NW = NC * NS
EPT = E // NW
CH = 80
NCHUNK = EPT // CH
RPT = NP // NS
ZR = 128
BM = 2000

_MESH = plsc.VectorSubcoreMesh(core_axis_name="c", subcore_axis_name="s")


def _deg_body(dst_hbm, out_hbm, dstv, onesv, zbuf, acc):
    c = lax.axis_index("c")
    s = lax.axis_index("s")
    wid = s * NC + c

    def fill(i, _):
        onesv[i] = jnp.ones((L,), jnp.float32)
        zbuf[i] = jnp.zeros((L,), jnp.float32)
        return 0

    lax.fori_loop(0, CH, fill, 0)

    def fillz(i, _):
        zbuf[i] = jnp.zeros((L,), jnp.float32)
        return 0

    lax.fori_loop(CH, ZR, fillz, 0)
    for j in range(RPT // ZR):
        pltpu.sync_copy(zbuf, acc.at[pl.ds(s * RPT + j * ZR, ZR)])
    plsc.subcore_barrier()

    def body(i, _):
        b = pl.multiple_of(wid * EPT + i * CH, 8)
        pltpu.sync_copy(dst_hbm.at[pl.ds(b, CH)], dstv)
        pltpu.sync_copy(onesv, acc.at[dstv], add=True)
        return 0

    lax.fori_loop(0, NCHUNK, body, 0)
    plsc.subcore_barrier()
    pltpu.sync_copy(acc.at[pl.ds(s * RPT, RPT)],
                    out_hbm.at[c, pl.ds(s * RPT, RPT)])


_sc_deg = pl.kernel(
    _deg_body,
    out_type=jax.ShapeDtypeStruct((NC, NP, L), jnp.float32),
    mesh=_MESH,
    scratch_types=[
        pltpu.VMEM((CH,), jnp.int32),
        pltpu.VMEM((CH, L), jnp.float32),
        pltpu.VMEM((ZR, L), jnp.float32),
        pltpu.VMEM_SHARED((NP, L), jnp.float32),
    ],
)


def _edge_body(g_hbm, src_hbm, dst_hbm, out_hbm, srcv, dstv, rows, zbuf, acc,
               sem):
    c = lax.axis_index("c")
    s = lax.axis_index("s")
    wid = s * NC + c

    def fillz(i, _):
        for jcol in range(H // L):
            zbuf[i, pl.ds(jcol * L, L)] = jnp.zeros((L,), jnp.float32)
        return 0

    lax.fori_loop(0, ZR, fillz, 0)
    for j in range(RPT // ZR):
        pltpu.sync_copy(zbuf, acc.at[pl.ds(s * RPT + j * ZR, ZR)])
    plsc.subcore_barrier()

    def body(i, _):
        b = pl.multiple_of(wid * EPT + i * CH, 8)
        pltpu.sync_copy(src_hbm.at[pl.ds(b, CH)], srcv)
        pltpu.sync_copy(dst_hbm.at[pl.ds(b, CH)], dstv)
        pltpu.async_copy(g_hbm.at[srcv], rows, sem).wait()
        pltpu.sync_copy(rows, acc.at[dstv], add=True)
        return 0

    lax.fori_loop(0, NCHUNK, body, 0)
    plsc.subcore_barrier()
    pltpu.sync_copy(acc.at[pl.ds(s * RPT, RPT)],
                    out_hbm.at[c, pl.ds(s * RPT, RPT)])


_sc_edge = pl.kernel(
    _edge_body,
    out_type=jax.ShapeDtypeStruct((NC, NP, H), jnp.float32),
    mesh=_MESH,
    scratch_types=[
        pltpu.VMEM((CH,), jnp.int32),
        pltpu.VMEM((CH,), jnp.int32),
        pltpu.VMEM((CH, H), jnp.float32),
        pltpu.VMEM((ZR, H), jnp.float32),
        pltpu.VMEM_SHARED((NP, H), jnp.float32),
        pltpu.SemaphoreType.DMA,
    ],
)


def _tc_l1_body(x_ref, w_ref, dacc_ref, g_ref, dis_ref):
    deg = dacc_ref[0, :, 0:1] + dacc_ref[1, :, 0:1] + 1.0
    dis = lax.rsqrt(deg)
    dis_ref[...] = dis
    h = lax.dot_general(x_ref[...], w_ref[...], (((1,), (1,)), ((), ())),
                        preferred_element_type=jnp.float32)
    g_ref[...] = h * dis


def _tc_l1(x, W1, dacc):
    return pl.pallas_call(
        _tc_l1_body,
        grid=(N // BM,),
        in_specs=[
            pl.BlockSpec((BM, D), lambda i: (i, 0)),
            pl.BlockSpec((H, D), lambda i: (0, 0)),
            pl.BlockSpec((NC, BM, L), lambda i: (0, i, 0)),
        ],
        out_specs=[
            pl.BlockSpec((BM, H), lambda i: (i, 0)),
            pl.BlockSpec((BM, 1), lambda i: (i, 0)),
        ],
        out_shape=[
            jax.ShapeDtypeStruct((N, H), jnp.float32),
            jax.ShapeDtypeStruct((N, 1), jnp.float32),
        ],
    )(x, W1, dacc)


def _tc_mid_body(acc_ref, g_ref, dis_ref, b_ref, w_ref, out_ref):
    dis = dis_ref[...]
    sfull = acc_ref[0] + acc_ref[1] + g_ref[...]
    a = jnp.maximum(sfull * dis + b_ref[...], 0.0)
    out_ref[...] = lax.dot_general(a, w_ref[...], (((1,), (1,)), ((), ())),
                                   preferred_element_type=jnp.float32) * dis


def _tc_mid(acc, g, dis, b, W):
    return pl.pallas_call(
        _tc_mid_body,
        grid=(N // BM,),
        in_specs=[
            pl.BlockSpec((NC, BM, H), lambda i: (0, i, 0)),
            pl.BlockSpec((BM, H), lambda i: (i, 0)),
            pl.BlockSpec((BM, 1), lambda i: (i, 0)),
            pl.BlockSpec((1, H), lambda i: (0, 0)),
            pl.BlockSpec((H, H), lambda i: (0, 0)),
        ],
        out_specs=pl.BlockSpec((BM, H), lambda i: (i, 0)),
        out_shape=jax.ShapeDtypeStruct((N, H), jnp.float32),
    )(acc, g, dis, b, W)


def _tc_fin_body(acc_ref, g_ref, dis_ref, b_ref, wl_ref, bl_ref, out_ref):
    sfull = acc_ref[0] + acc_ref[1] + g_ref[...]
    a = jnp.maximum(sfull * dis_ref[...] + b_ref[...], 0.0)
    out_ref[...] = jnp.sum(a * wl_ref[...], axis=1, keepdims=True) + bl_ref[...]


def _tc_fin(acc, g, dis, b7, Wl, bl):
    return pl.pallas_call(
        _tc_fin_body,
        grid=(N // BM,),
        in_specs=[
            pl.BlockSpec((NC, BM, H), lambda i: (0, i, 0)),
            pl.BlockSpec((BM, H), lambda i: (i, 0)),
            pl.BlockSpec((BM, 1), lambda i: (i, 0)),
            pl.BlockSpec((1, H), lambda i: (0, 0)),
            pl.BlockSpec((1, H), lambda i: (0, 0)),
            pl.BlockSpec((1, 1), lambda i: (0, 0)),
        ],
        out_specs=pl.BlockSpec((BM, 1), lambda i: (i, 0)),
        out_shape=jax.ShapeDtypeStruct((N, 1), jnp.float32),
    )(acc, g, dis, b7, Wl, bl)


def kernel(x, edge_index, W1, b1, W2, b2, W3, b3, W4, b4, W5, b5, W6, b6,
           W7, b7, Wl, bl):
    src = edge_index[0]
    dst = edge_index[1]
    dacc = _sc_deg(dst)
    g, dis = _tc_l1(x, W1, dacc)
    for Wn, bn in ((W2, b1), (W3, b2), (W4, b3), (W5, b4), (W6, b5),
                   (W7, b6)):
        acc = _sc_edge(g, src, dst)
        g = _tc_mid(acc, g, dis, bn.reshape(1, H), Wn)
    acc = _sc_edge(g, src, dst)
    out = _tc_fin(acc, g, dis, b7.reshape(1, H), Wl, bl.reshape(1, 1))
    return out.reshape(N)


# speedup vs baseline: 10.8096x; 1.2049x over previous
import jax
import jax.numpy as jnp
from jax import lax
from jax.experimental import pallas as pl
from jax.experimental.pallas import tpu as pltpu
from jax.experimental.pallas import tpu_sc as plsc

N, D, H = 10000, 128, 128
E = 320000
NP = 10240
NC, NS, L = 2, 16, 16
NW = NC * NS
EPT = E // NW
CH = 80
NCHUNK = EPT // CH
ECH = 96
NFULL = EPT // ECH
REM = EPT - NFULL * ECH
RPT = NP // NS
ZR = 128
BM = 2000

_MESH = plsc.VectorSubcoreMesh(core_axis_name="c", subcore_axis_name="s")


def _deg_body(dst_hbm, out_hbm, dstv, onesv, zbuf, acc):
    c = lax.axis_index("c")
    s = lax.axis_index("s")
    wid = s * NC + c

    def fill(i, _):
        onesv[i] = jnp.ones((L,), jnp.float32)
        zbuf[i] = jnp.zeros((L,), jnp.float32)
        return 0

    lax.fori_loop(0, CH, fill, 0)

    def fillz(i, _):
        zbuf[i] = jnp.zeros((L,), jnp.float32)
        return 0

    lax.fori_loop(CH, ZR, fillz, 0)
    for j in range(RPT // ZR):
        pltpu.sync_copy(zbuf, acc.at[pl.ds(s * RPT + j * ZR, ZR)])
    plsc.subcore_barrier()

    def body(i, _):
        b = pl.multiple_of(wid * EPT + i * CH, 8)
        pltpu.sync_copy(dst_hbm.at[pl.ds(b, CH)], dstv)
        pltpu.sync_copy(onesv, acc.at[dstv], add=True)
        return 0

    lax.fori_loop(0, NCHUNK, body, 0)
    plsc.subcore_barrier()
    pltpu.sync_copy(acc.at[pl.ds(s * RPT, RPT)],
                    out_hbm.at[c, pl.ds(s * RPT, RPT)])


_sc_deg = pl.kernel(
    _deg_body,
    out_type=jax.ShapeDtypeStruct((NC, NP, L), jnp.float32),
    mesh=_MESH,
    scratch_types=[
        pltpu.VMEM((CH,), jnp.int32),
        pltpu.VMEM((CH, L), jnp.float32),
        pltpu.VMEM((ZR, L), jnp.float32),
        pltpu.VMEM_SHARED((NP, L), jnp.float32),
    ],
)


NCHT = -(-EPT // ECH)


def _edge_body(g_hbm, src2_hbm, dst2_hbm, out_hbm,
               srcv, dstv, rows0, rows1, acc, sem):
    c = lax.axis_index("c")
    s = lax.axis_index("s")
    wid = s * NC + c

    def fillz(i, _):
        for jcol in range(H // L):
            rows0[i, pl.ds(jcol * L, L)] = jnp.zeros((L,), jnp.float32)
        return 0

    lax.fori_loop(0, 64, fillz, 0)
    for j in range(RPT // 64):
        pltpu.sync_copy(rows0.at[pl.ds(0, 64)],
                        acc.at[pl.ds(s * RPT + j * 64, 64)])
    pltpu.sync_copy(src2_hbm.at[wid], srcv)
    pltpu.sync_copy(dst2_hbm.at[wid], dstv)
    plsc.subcore_barrier()

    def body(j, _):
        i0 = 2 * j
        pltpu.async_copy(g_hbm.at[srcv.at[pl.ds(i0 * ECH, ECH)]], rows0, sem)
        pltpu.async_copy(g_hbm.at[srcv.at[pl.ds(i0 * ECH + ECH, ECH)]], rows1, sem)
        pltpu.make_async_copy(g_hbm.at[srcv.at[pl.ds(i0 * ECH, ECH)]], rows0, sem).wait()
        pltpu.make_async_copy(g_hbm.at[srcv.at[pl.ds(i0 * ECH + ECH, ECH)]], rows1, sem).wait()
        pltpu.sync_copy(rows0, acc.at[dstv.at[i0]], add=True)
        pltpu.sync_copy(rows1, acc.at[dstv.at[i0 + 1]], add=True)
        return 0

    lax.fori_loop(0, NCHT // 2, body, 0)
    it = NCHT - 1
    pltpu.async_copy(g_hbm.at[srcv.at[pl.ds(it * ECH, ECH)]], rows0, sem).wait()
    pltpu.sync_copy(rows0, acc.at[dstv.at[it]], add=True)
    plsc.subcore_barrier()
    pltpu.sync_copy(acc.at[pl.ds(s * RPT, RPT)],
                    out_hbm.at[c, pl.ds(s * RPT, RPT)])


_sc_edge = pl.kernel(
    _edge_body,
    out_type=jax.ShapeDtypeStruct((NC, NP, H), jnp.float32),
    mesh=_MESH,
    scratch_types=[
        pltpu.VMEM((NCHT * ECH,), jnp.int32),
        pltpu.VMEM((NCHT, ECH), jnp.int32),
        pltpu.VMEM((ECH, H), jnp.float32),
        pltpu.VMEM((ECH, H), jnp.float32),
        pltpu.VMEM_SHARED((NP, H), jnp.float32),
        pltpu.SemaphoreType.DMA,
    ],
)


def _tc_l1_body(x_ref, w_ref, dacc_ref, g_ref, dis_ref):
    deg = dacc_ref[0, :, 0:1] + dacc_ref[1, :, 0:1] + 1.0
    dis = lax.rsqrt(deg)
    dis_ref[...] = dis
    h = lax.dot_general(x_ref[...], w_ref[...], (((1,), (1,)), ((), ())),
                        preferred_element_type=jnp.float32)
    g_ref[...] = h * dis


def _tc_l1(x, W1, dacc):
    return pl.pallas_call(
        _tc_l1_body,
        grid=(N // BM,),
        in_specs=[
            pl.BlockSpec((BM, D), lambda i: (i, 0)),
            pl.BlockSpec((H, D), lambda i: (0, 0)),
            pl.BlockSpec((NC, BM, L), lambda i: (0, i, 0)),
        ],
        out_specs=[
            pl.BlockSpec((BM, H), lambda i: (i, 0)),
            pl.BlockSpec((BM, 1), lambda i: (i, 0)),
        ],
        out_shape=[
            jax.ShapeDtypeStruct((N, H), jnp.float32),
            jax.ShapeDtypeStruct((N, 1), jnp.float32),
        ],
    )(x, W1, dacc)


def _tc_mid_body(acc_ref, g_ref, dis_ref, b_ref, w_ref, out_ref):
    dis = dis_ref[...]
    sfull = acc_ref[0] + acc_ref[1] + g_ref[...]
    a = jnp.maximum(sfull * dis + b_ref[...], 0.0)
    out_ref[...] = lax.dot_general(a, w_ref[...], (((1,), (1,)), ((), ())),
                                   preferred_element_type=jnp.float32) * dis


def _tc_mid(acc, g, dis, b, W):
    return pl.pallas_call(
        _tc_mid_body,
        grid=(N // BM,),
        in_specs=[
            pl.BlockSpec((NC, BM, H), lambda i: (0, i, 0)),
            pl.BlockSpec((BM, H), lambda i: (i, 0)),
            pl.BlockSpec((BM, 1), lambda i: (i, 0)),
            pl.BlockSpec((1, H), lambda i: (0, 0)),
            pl.BlockSpec((H, H), lambda i: (0, 0)),
        ],
        out_specs=pl.BlockSpec((BM, H), lambda i: (i, 0)),
        out_shape=jax.ShapeDtypeStruct((N, H), jnp.float32),
    )(acc, g, dis, b, W)


def _tc_fin_body(acc_ref, g_ref, dis_ref, b_ref, wl_ref, bl_ref, out_ref):
    sfull = acc_ref[0] + acc_ref[1] + g_ref[...]
    a = jnp.maximum(sfull * dis_ref[...] + b_ref[...], 0.0)
    out_ref[...] = jnp.sum(a * wl_ref[...], axis=1, keepdims=True) + bl_ref[...]


def _tc_fin(acc, g, dis, b7, Wl, bl):
    return pl.pallas_call(
        _tc_fin_body,
        grid=(N // BM,),
        in_specs=[
            pl.BlockSpec((NC, BM, H), lambda i: (0, i, 0)),
            pl.BlockSpec((BM, H), lambda i: (i, 0)),
            pl.BlockSpec((BM, 1), lambda i: (i, 0)),
            pl.BlockSpec((1, H), lambda i: (0, 0)),
            pl.BlockSpec((1, H), lambda i: (0, 0)),
            pl.BlockSpec((1, 1), lambda i: (0, 0)),
        ],
        out_specs=pl.BlockSpec((BM, 1), lambda i: (i, 0)),
        out_shape=jax.ShapeDtypeStruct((N, 1), jnp.float32),
    )(acc, g, dis, b7, Wl, bl)


def kernel(x, edge_index, W1, b1, W2, b2, W3, b3, W4, b4, W5, b5, W6, b6,
           W7, b7, Wl, bl):
    src = edge_index[0]
    dst = edge_index[1]
    src2 = jnp.pad(src.reshape(NW, EPT), ((0, 0), (0, NCHT * ECH - EPT)),
                   constant_values=0)
    dst2 = jnp.pad(dst.reshape(NW, EPT), ((0, 0), (0, NCHT * ECH - EPT)),
                   constant_values=NP - 1).reshape(NW, NCHT, ECH)
    dacc = _sc_deg(dst)
    g, dis = _tc_l1(x, W1, dacc)
    for Wn, bn in ((W2, b1), (W3, b2), (W4, b3), (W5, b4), (W6, b5),
                   (W7, b6)):
        acc = _sc_edge(g, src2, dst2)
        g = _tc_mid(acc, g, dis, bn.reshape(1, H), Wn)
    acc = _sc_edge(g, src2, dst2)
    out = _tc_fin(acc, g, dis, b7.reshape(1, H), Wl, bl.reshape(1, 1))
    return out.reshape(N)


# speedup vs baseline: 11.2717x; 1.0427x over previous
import jax
import jax.numpy as jnp
from jax import lax
from jax.experimental import pallas as pl
from jax.experimental.pallas import tpu as pltpu
from jax.experimental.pallas import tpu_sc as plsc

N, D, H = 10000, 128, 128
E = 320000
NP = 10240
NC, NS, L = 2, 16, 16
NW = NC * NS
EPT = E // NW
CH = 80
NCHUNK = EPT // CH
ECH = 96
NFULL = EPT // ECH
REM = EPT - NFULL * ECH
NCHT = -(-EPT // ECH)
RPT = NP // NS
ZR = 128
BM = 2000

_MESH = plsc.VectorSubcoreMesh(core_axis_name="c", subcore_axis_name="s")


def _deg_body(dst2_hbm, out_hbm, dstv, onesv, zbuf, acc, sem):
    c = lax.axis_index("c")
    s = lax.axis_index("s")
    wid = s * NC + c

    def fill(i, _):
        onesv[i] = jnp.ones((L,), jnp.float32)
        return 0

    lax.fori_loop(0, ECH, fill, 0)

    def fillz(i, _):
        zbuf[i] = jnp.zeros((L,), jnp.float32)
        return 0

    lax.fori_loop(0, 64, fillz, 0)
    for j in range(RPT // 64):
        pltpu.sync_copy(zbuf, acc.at[pl.ds(s * RPT + j * 64, 64)])
    pltpu.sync_copy(dst2_hbm.at[wid], dstv)
    plsc.subcore_barrier()

    def body(j, _):
        i0 = 2 * j
        pltpu.async_copy(onesv, acc.at[dstv.at[i0]], sem, add=True)
        pltpu.async_copy(onesv, acc.at[dstv.at[i0 + 1]], sem, add=True)
        pltpu.make_async_copy(onesv, acc.at[dstv.at[i0]], sem).wait()
        pltpu.make_async_copy(onesv, acc.at[dstv.at[i0 + 1]], sem).wait()
        return 0

    lax.fori_loop(0, NCHT // 2, body, 0)
    pltpu.sync_copy(onesv, acc.at[dstv.at[NCHT - 1]], add=True)
    plsc.subcore_barrier()
    pltpu.sync_copy(acc.at[pl.ds(s * RPT, RPT)],
                    out_hbm.at[c, pl.ds(s * RPT, RPT)])


_sc_deg = pl.kernel(
    _deg_body,
    out_type=jax.ShapeDtypeStruct((NC, NP, L), jnp.float32),
    mesh=_MESH,
    scratch_types=[
        pltpu.VMEM((NCHT, ECH), jnp.int32),
        pltpu.VMEM((ECH, L), jnp.float32),
        pltpu.VMEM((64, L), jnp.float32),
        pltpu.VMEM_SHARED((NP, L), jnp.float32),
        pltpu.SemaphoreType.DMA,
    ],
)


def _edge_body(g_hbm, src2_hbm, dst2_hbm, out_hbm,
               srcv, dstv, rows0, rows1, acc, sem, sems):
    c = lax.axis_index("c")
    s = lax.axis_index("s")
    wid = s * NC + c

    def fillz(i, _):
        for jcol in range(H // L):
            rows0[i, pl.ds(jcol * L, L)] = jnp.zeros((L,), jnp.float32)
        return 0

    lax.fori_loop(0, 64, fillz, 0)
    for j in range(RPT // 64):
        pltpu.sync_copy(rows0.at[pl.ds(0, 64)],
                        acc.at[pl.ds(s * RPT + j * 64, 64)])
    pltpu.sync_copy(src2_hbm.at[wid], srcv)
    pltpu.sync_copy(dst2_hbm.at[wid], dstv)
    plsc.subcore_barrier()

    def body(j, _):
        i0 = 2 * j
        pltpu.async_copy(g_hbm.at[srcv.at[pl.ds(i0 * ECH, ECH)]], rows0, sem)
        pltpu.async_copy(g_hbm.at[srcv.at[pl.ds(i0 * ECH + ECH, ECH)]], rows1, sem)
        pltpu.make_async_copy(g_hbm.at[srcv.at[pl.ds(i0 * ECH, ECH)]], rows0, sem).wait()
        pltpu.make_async_copy(g_hbm.at[srcv.at[pl.ds(i0 * ECH + ECH, ECH)]], rows1, sem).wait()
        pltpu.async_copy(rows0, acc.at[dstv.at[i0]], sems, add=True)
        pltpu.async_copy(rows1, acc.at[dstv.at[i0 + 1]], sems, add=True)
        pltpu.make_async_copy(rows0, acc.at[dstv.at[i0]], sems).wait()
        pltpu.make_async_copy(rows1, acc.at[dstv.at[i0 + 1]], sems).wait()
        return 0

    lax.fori_loop(0, NCHT // 2, body, 0)
    it = NCHT - 1
    pltpu.async_copy(g_hbm.at[srcv.at[pl.ds(it * ECH, ECH)]], rows0, sem).wait()
    pltpu.sync_copy(rows0, acc.at[dstv.at[it]], add=True)
    plsc.subcore_barrier()
    pltpu.sync_copy(acc.at[pl.ds(s * RPT, RPT)],
                    out_hbm.at[c, pl.ds(s * RPT, RPT)])


_sc_edge = pl.kernel(
    _edge_body,
    out_type=jax.ShapeDtypeStruct((NC, NP, H), jnp.float32),
    mesh=_MESH,
    scratch_types=[
        pltpu.VMEM((NCHT * ECH,), jnp.int32),
        pltpu.VMEM((NCHT, ECH), jnp.int32),
        pltpu.VMEM((ECH, H), jnp.float32),
        pltpu.VMEM((ECH, H), jnp.float32),
        pltpu.VMEM_SHARED((NP, H), jnp.float32),
        pltpu.SemaphoreType.DMA,
        pltpu.SemaphoreType.DMA,
    ],
)


def _tc_l1_body(x_ref, w_ref, dacc_ref, g_ref, dis_ref):
    deg = dacc_ref[0, :, 0:1] + dacc_ref[1, :, 0:1] + 1.0
    dis = lax.rsqrt(deg)
    dis_ref[...] = dis
    h = lax.dot_general(x_ref[...], w_ref[...], (((1,), (1,)), ((), ())),
                        preferred_element_type=jnp.float32)
    g_ref[...] = h * dis


def _tc_l1(x, W1, dacc):
    return pl.pallas_call(
        _tc_l1_body,
        grid=(N // BM,),
        in_specs=[
            pl.BlockSpec((BM, D), lambda i: (i, 0)),
            pl.BlockSpec((H, D), lambda i: (0, 0)),
            pl.BlockSpec((NC, BM, L), lambda i: (0, i, 0)),
        ],
        out_specs=[
            pl.BlockSpec((BM, H), lambda i: (i, 0)),
            pl.BlockSpec((BM, 1), lambda i: (i, 0)),
        ],
        out_shape=[
            jax.ShapeDtypeStruct((N, H), jnp.float32),
            jax.ShapeDtypeStruct((N, 1), jnp.float32),
        ],
    )(x, W1, dacc)


def _tc_mid_body(acc_ref, g_ref, dis_ref, b_ref, w_ref, out_ref):
    dis = dis_ref[...]
    sfull = acc_ref[0] + acc_ref[1] + g_ref[...]
    a = jnp.maximum(sfull * dis + b_ref[...], 0.0)
    out_ref[...] = lax.dot_general(a, w_ref[...], (((1,), (1,)), ((), ())),
                                   preferred_element_type=jnp.float32) * dis


def _tc_mid(acc, g, dis, b, W):
    return pl.pallas_call(
        _tc_mid_body,
        grid=(N // BM,),
        in_specs=[
            pl.BlockSpec((NC, BM, H), lambda i: (0, i, 0)),
            pl.BlockSpec((BM, H), lambda i: (i, 0)),
            pl.BlockSpec((BM, 1), lambda i: (i, 0)),
            pl.BlockSpec((1, H), lambda i: (0, 0)),
            pl.BlockSpec((H, H), lambda i: (0, 0)),
        ],
        out_specs=pl.BlockSpec((BM, H), lambda i: (i, 0)),
        out_shape=jax.ShapeDtypeStruct((N, H), jnp.float32),
    )(acc, g, dis, b, W)


def _tc_fin_body(acc_ref, g_ref, dis_ref, b_ref, wl_ref, bl_ref, out_ref):
    sfull = acc_ref[0] + acc_ref[1] + g_ref[...]
    a = jnp.maximum(sfull * dis_ref[...] + b_ref[...], 0.0)
    out_ref[...] = jnp.sum(a * wl_ref[...], axis=1, keepdims=True) + bl_ref[...]


def _tc_fin(acc, g, dis, b7, Wl, bl):
    return pl.pallas_call(
        _tc_fin_body,
        grid=(N // BM,),
        in_specs=[
            pl.BlockSpec((NC, BM, H), lambda i: (0, i, 0)),
            pl.BlockSpec((BM, H), lambda i: (i, 0)),
            pl.BlockSpec((BM, 1), lambda i: (i, 0)),
            pl.BlockSpec((1, H), lambda i: (0, 0)),
            pl.BlockSpec((1, H), lambda i: (0, 0)),
            pl.BlockSpec((1, 1), lambda i: (0, 0)),
        ],
        out_specs=pl.BlockSpec((BM, 1), lambda i: (i, 0)),
        out_shape=jax.ShapeDtypeStruct((N, 1), jnp.float32),
    )(acc, g, dis, b7, Wl, bl)


def kernel(x, edge_index, W1, b1, W2, b2, W3, b3, W4, b4, W5, b5, W6, b6,
           W7, b7, Wl, bl):
    src = edge_index[0]
    dst = edge_index[1]
    src2 = jnp.pad(src.reshape(NW, EPT), ((0, 0), (0, NCHT * ECH - EPT)),
                   constant_values=0)
    dst2 = jnp.pad(dst.reshape(NW, EPT), ((0, 0), (0, NCHT * ECH - EPT)),
                   constant_values=NP - 1).reshape(NW, NCHT, ECH)
    dacc = _sc_deg(dst2)
    g, dis = _tc_l1(x, W1, dacc)
    for Wn, bn in ((W2, b1), (W3, b2), (W4, b3), (W5, b4), (W6, b5),
                   (W7, b6)):
        acc = _sc_edge(g, src2, dst2)
        g = _tc_mid(acc, g, dis, bn.reshape(1, H), Wn)
    acc = _sc_edge(g, src2, dst2)
    out = _tc_fin(acc, g, dis, b7.reshape(1, H), Wl, bl.reshape(1, 1))
    return out.reshape(N)


# speedup vs baseline: 11.2793x; 1.0007x over previous
import jax
import jax.numpy as jnp
from jax import lax
from jax.experimental import pallas as pl
from jax.experimental.pallas import tpu as pltpu
from jax.experimental.pallas import tpu_sc as plsc

N, D, H = 10000, 128, 128
E = 320000
NP = 10240
NC, NS, L = 2, 16, 16
NW = NC * NS
EPT = E // NW
CH = 80
NCHUNK = EPT // CH
ECH = 96
NFULL = EPT // ECH
REM = EPT - NFULL * ECH
NCHT = -(-EPT // ECH)
RPT = NP // NS
ZR = 128
BM = 2000

_MESH = plsc.VectorSubcoreMesh(core_axis_name="c", subcore_axis_name="s")


def _deg_body(dst2_hbm, out_hbm, dstv, onesv, zbuf, acc, sem):
    c = lax.axis_index("c")
    s = lax.axis_index("s")
    wid = s * NC + c

    def fill(i, _):
        onesv[i] = jnp.ones((L,), jnp.float32)
        return 0

    lax.fori_loop(0, ECH, fill, 0)

    def fillz(i, _):
        zbuf[i] = jnp.zeros((L,), jnp.float32)
        return 0

    lax.fori_loop(0, 64, fillz, 0)
    for j in range(RPT // 64):
        pltpu.sync_copy(zbuf, acc.at[pl.ds(s * RPT + j * 64, 64)])
    pltpu.sync_copy(dst2_hbm.at[wid], dstv)
    plsc.subcore_barrier()

    def body(j, _):
        i0 = 2 * j
        pltpu.async_copy(onesv, acc.at[dstv.at[pl.ds(i0 * ECH, ECH)]], sem, add=True)
        pltpu.async_copy(onesv, acc.at[dstv.at[pl.ds(i0 * ECH + ECH, ECH)]], sem, add=True)
        pltpu.make_async_copy(onesv, acc.at[dstv.at[pl.ds(i0 * ECH, ECH)]], sem).wait()
        pltpu.make_async_copy(onesv, acc.at[dstv.at[pl.ds(i0 * ECH + ECH, ECH)]], sem).wait()
        return 0

    lax.fori_loop(0, NCHT // 2, body, 0)
    pltpu.sync_copy(onesv, acc.at[dstv.at[pl.ds((NCHT - 1) * ECH, ECH)]], add=True)
    plsc.subcore_barrier()
    pltpu.sync_copy(acc.at[pl.ds(s * RPT, RPT)],
                    out_hbm.at[c, pl.ds(s * RPT, RPT)])


_sc_deg = pl.kernel(
    _deg_body,
    out_type=jax.ShapeDtypeStruct((NC, NP, L), jnp.float32),
    mesh=_MESH,
    scratch_types=[
        pltpu.VMEM((NCHT * ECH,), jnp.int32),
        pltpu.VMEM((ECH, L), jnp.float32),
        pltpu.VMEM((64, L), jnp.float32),
        pltpu.VMEM_SHARED((NP, L), jnp.float32),
        pltpu.SemaphoreType.DMA,
    ],
)


def _edge_body(g_hbm, src2_hbm, dst2_hbm, out_hbm,
               srcv, dstv, rows0, rows1, acc, sem, sems):
    c = lax.axis_index("c")
    s = lax.axis_index("s")
    wid = s * NC + c

    def fillz(i, _):
        for jcol in range(H // L):
            rows0[i, pl.ds(jcol * L, L)] = jnp.zeros((L,), jnp.float32)
        return 0

    lax.fori_loop(0, 64, fillz, 0)
    for j in range(RPT // 64):
        pltpu.sync_copy(rows0.at[pl.ds(0, 64)],
                        acc.at[pl.ds(s * RPT + j * 64, 64)])
    pltpu.sync_copy(src2_hbm.at[wid], srcv)
    pltpu.sync_copy(dst2_hbm.at[wid], dstv)
    plsc.subcore_barrier()

    def body(j, _):
        i0 = 2 * j
        pltpu.async_copy(g_hbm.at[srcv.at[pl.ds(i0 * ECH, ECH)]], rows0, sem)
        pltpu.async_copy(g_hbm.at[srcv.at[pl.ds(i0 * ECH + ECH, ECH)]], rows1, sem)
        pltpu.make_async_copy(g_hbm.at[srcv.at[pl.ds(i0 * ECH, ECH)]], rows0, sem).wait()
        pltpu.make_async_copy(g_hbm.at[srcv.at[pl.ds(i0 * ECH + ECH, ECH)]], rows1, sem).wait()
        pltpu.async_copy(rows0, acc.at[dstv.at[pl.ds(i0 * ECH, ECH)]], sems, add=True)
        pltpu.async_copy(rows1, acc.at[dstv.at[pl.ds(i0 * ECH + ECH, ECH)]], sems, add=True)
        pltpu.make_async_copy(rows0, acc.at[dstv.at[pl.ds(i0 * ECH, ECH)]], sems).wait()
        pltpu.make_async_copy(rows1, acc.at[dstv.at[pl.ds(i0 * ECH + ECH, ECH)]], sems).wait()
        return 0

    lax.fori_loop(0, NCHT // 2, body, 0)
    it = NCHT - 1
    pltpu.async_copy(g_hbm.at[srcv.at[pl.ds(it * ECH, ECH)]], rows0, sem).wait()
    pltpu.sync_copy(rows0, acc.at[dstv.at[pl.ds(it * ECH, ECH)]], add=True)
    plsc.subcore_barrier()
    pltpu.sync_copy(acc.at[pl.ds(s * RPT, RPT)],
                    out_hbm.at[c, pl.ds(s * RPT, RPT)])


_sc_edge = pl.kernel(
    _edge_body,
    out_type=jax.ShapeDtypeStruct((NC, NP, H), jnp.float32),
    mesh=_MESH,
    scratch_types=[
        pltpu.VMEM((NCHT * ECH,), jnp.int32),
        pltpu.VMEM((NCHT * ECH,), jnp.int32),
        pltpu.VMEM((ECH, H), jnp.float32),
        pltpu.VMEM((ECH, H), jnp.float32),
        pltpu.VMEM_SHARED((NP, H), jnp.float32),
        pltpu.SemaphoreType.DMA,
        pltpu.SemaphoreType.DMA,
    ],
)


def _tc_l1_body(x_ref, w_ref, dacc_ref, g_ref, dis_ref):
    deg = dacc_ref[0, :, 0:1] + dacc_ref[1, :, 0:1] + 1.0
    dis = lax.rsqrt(deg)
    dis_ref[...] = dis
    h = lax.dot_general(x_ref[...], w_ref[...], (((1,), (1,)), ((), ())),
                        preferred_element_type=jnp.float32)
    g_ref[...] = h * dis


def _tc_l1(x, W1, dacc):
    return pl.pallas_call(
        _tc_l1_body,
        grid=(N // BM,),
        in_specs=[
            pl.BlockSpec((BM, D), lambda i: (i, 0)),
            pl.BlockSpec((H, D), lambda i: (0, 0)),
            pl.BlockSpec((NC, BM, L), lambda i: (0, i, 0)),
        ],
        out_specs=[
            pl.BlockSpec((BM, H), lambda i: (i, 0)),
            pl.BlockSpec((BM, 1), lambda i: (i, 0)),
        ],
        out_shape=[
            jax.ShapeDtypeStruct((N, H), jnp.float32),
            jax.ShapeDtypeStruct((N, 1), jnp.float32),
        ],
    )(x, W1, dacc)


def _tc_mid_body(acc_ref, g_ref, dis_ref, b_ref, w_ref, out_ref):
    dis = dis_ref[...]
    sfull = acc_ref[0] + acc_ref[1] + g_ref[...]
    a = jnp.maximum(sfull * dis + b_ref[...], 0.0)
    out_ref[...] = lax.dot_general(a, w_ref[...], (((1,), (1,)), ((), ())),
                                   preferred_element_type=jnp.float32) * dis


def _tc_mid(acc, g, dis, b, W):
    return pl.pallas_call(
        _tc_mid_body,
        grid=(N // BM,),
        in_specs=[
            pl.BlockSpec((NC, BM, H), lambda i: (0, i, 0)),
            pl.BlockSpec((BM, H), lambda i: (i, 0)),
            pl.BlockSpec((BM, 1), lambda i: (i, 0)),
            pl.BlockSpec((1, H), lambda i: (0, 0)),
            pl.BlockSpec((H, H), lambda i: (0, 0)),
        ],
        out_specs=pl.BlockSpec((BM, H), lambda i: (i, 0)),
        out_shape=jax.ShapeDtypeStruct((N, H), jnp.float32),
    )(acc, g, dis, b, W)


def _tc_fin_body(acc_ref, g_ref, dis_ref, b_ref, wl_ref, bl_ref, out_ref):
    sfull = acc_ref[0] + acc_ref[1] + g_ref[...]
    a = jnp.maximum(sfull * dis_ref[...] + b_ref[...], 0.0)
    out_ref[...] = jnp.sum(a * wl_ref[...], axis=1, keepdims=True) + bl_ref[...]


def _tc_fin(acc, g, dis, b7, Wl, bl):
    return pl.pallas_call(
        _tc_fin_body,
        grid=(N // BM,),
        in_specs=[
            pl.BlockSpec((NC, BM, H), lambda i: (0, i, 0)),
            pl.BlockSpec((BM, H), lambda i: (i, 0)),
            pl.BlockSpec((BM, 1), lambda i: (i, 0)),
            pl.BlockSpec((1, H), lambda i: (0, 0)),
            pl.BlockSpec((1, H), lambda i: (0, 0)),
            pl.BlockSpec((1, 1), lambda i: (0, 0)),
        ],
        out_specs=pl.BlockSpec((BM, 1), lambda i: (i, 0)),
        out_shape=jax.ShapeDtypeStruct((N, 1), jnp.float32),
    )(acc, g, dis, b7, Wl, bl)


def kernel(x, edge_index, W1, b1, W2, b2, W3, b3, W4, b4, W5, b5, W6, b6,
           W7, b7, Wl, bl):
    src = edge_index[0]
    dst = edge_index[1]
    src2 = jnp.pad(src.reshape(NW, EPT), ((0, 0), (0, NCHT * ECH - EPT)),
                   constant_values=0)
    dst2 = jnp.pad(dst.reshape(NW, EPT), ((0, 0), (0, NCHT * ECH - EPT)),
                   constant_values=NP - 1)
    dacc = _sc_deg(dst2)
    g, dis = _tc_l1(x, W1, dacc)
    for Wn, bn in ((W2, b1), (W3, b2), (W4, b3), (W5, b4), (W6, b5),
                   (W7, b6)):
        acc = _sc_edge(g, src2, dst2)
        g = _tc_mid(acc, g, dis, bn.reshape(1, H), Wn)
    acc = _sc_edge(g, src2, dst2)
    out = _tc_fin(acc, g, dis, b7.reshape(1, H), Wl, bl.reshape(1, 1))
    return out.reshape(N)


# speedup vs baseline: 11.4666x; 1.0166x over previous
import jax
import jax.numpy as jnp
from jax import lax
from jax.experimental import pallas as pl
from jax.experimental.pallas import tpu as pltpu
from jax.experimental.pallas import tpu_sc as plsc

N, D, H = 10000, 128, 128
E = 320000
NP = 10112
NC, NS, L = 2, 16, 16
NW = NC * NS
EPT = E // NW
CH = 80
NCHUNK = EPT // CH
ECH = 112
NFULL = EPT // ECH
REM = EPT - NFULL * ECH
NCHT = 90
RPT = NP // NS
ZR = 128
BM = 2000

_MESH = plsc.VectorSubcoreMesh(core_axis_name="c", subcore_axis_name="s")


def _deg_body(dst2_hbm, out_hbm, dstv, onesv, zbuf, acc, sem):
    c = lax.axis_index("c")
    s = lax.axis_index("s")
    wid = s * NC + c

    def fill(i, _):
        onesv[i] = jnp.ones((L,), jnp.float32)
        return 0

    lax.fori_loop(0, ECH, fill, 0)

    def fillz(i, _):
        zbuf[i] = jnp.zeros((L,), jnp.float32)
        return 0

    lax.fori_loop(0, 79, fillz, 0)
    for j in range(RPT // 79):
        pltpu.sync_copy(zbuf, acc.at[pl.ds(s * RPT + j * 79, 79)])
    pltpu.sync_copy(dst2_hbm.at[wid], dstv)
    plsc.subcore_barrier()

    def body(j, _):
        i0 = 2 * j
        pltpu.async_copy(onesv, acc.at[dstv.at[pl.ds(i0 * ECH, ECH)]], sem, add=True)
        pltpu.async_copy(onesv, acc.at[dstv.at[pl.ds(i0 * ECH + ECH, ECH)]], sem, add=True)
        pltpu.make_async_copy(onesv, acc.at[dstv.at[pl.ds(i0 * ECH, ECH)]], sem).wait()
        pltpu.make_async_copy(onesv, acc.at[dstv.at[pl.ds(i0 * ECH + ECH, ECH)]], sem).wait()
        return 0

    lax.fori_loop(0, NCHT // 2, body, 0)
    plsc.subcore_barrier()
    pltpu.sync_copy(acc.at[pl.ds(s * RPT, RPT)],
                    out_hbm.at[c, pl.ds(s * RPT, RPT)])


_sc_deg = pl.kernel(
    _deg_body,
    out_type=jax.ShapeDtypeStruct((NC, NP, L), jnp.float32),
    mesh=_MESH,
    scratch_types=[
        pltpu.VMEM((NCHT * ECH,), jnp.int32),
        pltpu.VMEM((ECH, L), jnp.float32),
        pltpu.VMEM((79, L), jnp.float32),
        pltpu.VMEM_SHARED((NP, L), jnp.float32),
        pltpu.SemaphoreType.DMA,
    ],
)


def _edge_body(g_hbm, src2_hbm, dst2_hbm, out_hbm,
               srcv, dstv, rows0, rows1, acc, sem, sems):
    c = lax.axis_index("c")
    s = lax.axis_index("s")
    wid = s * NC + c

    def fillz(i, _):
        for jcol in range(H // L):
            rows0[i, pl.ds(jcol * L, L)] = jnp.zeros((L,), jnp.float32)
        return 0

    lax.fori_loop(0, 79, fillz, 0)
    for j in range(RPT // 79):
        pltpu.sync_copy(rows0.at[pl.ds(0, 79)],
                        acc.at[pl.ds(s * RPT + j * 79, 79)])
    pltpu.sync_copy(src2_hbm.at[wid], srcv)
    pltpu.sync_copy(dst2_hbm.at[wid], dstv)
    plsc.subcore_barrier()

    def body(j, _):
        i0 = 2 * j
        pltpu.async_copy(g_hbm.at[srcv.at[pl.ds(i0 * ECH, ECH)]], rows0, sem)
        pltpu.async_copy(g_hbm.at[srcv.at[pl.ds(i0 * ECH + ECH, ECH)]], rows1, sem)
        pltpu.make_async_copy(g_hbm.at[srcv.at[pl.ds(i0 * ECH, ECH)]], rows0, sem).wait()
        pltpu.make_async_copy(g_hbm.at[srcv.at[pl.ds(i0 * ECH + ECH, ECH)]], rows1, sem).wait()
        pltpu.async_copy(rows0, acc.at[dstv.at[pl.ds(i0 * ECH, ECH)]], sems, add=True)
        pltpu.async_copy(rows1, acc.at[dstv.at[pl.ds(i0 * ECH + ECH, ECH)]], sems, add=True)
        pltpu.make_async_copy(rows0, acc.at[dstv.at[pl.ds(i0 * ECH, ECH)]], sems).wait()
        pltpu.make_async_copy(rows1, acc.at[dstv.at[pl.ds(i0 * ECH + ECH, ECH)]], sems).wait()
        return 0

    lax.fori_loop(0, NCHT // 2, body, 0)
    plsc.subcore_barrier()
    pltpu.sync_copy(acc.at[pl.ds(s * RPT, RPT)],
                    out_hbm.at[c, pl.ds(s * RPT, RPT)])


_sc_edge = pl.kernel(
    _edge_body,
    out_type=jax.ShapeDtypeStruct((NC, NP, H), jnp.float32),
    mesh=_MESH,
    scratch_types=[
        pltpu.VMEM((NCHT * ECH,), jnp.int32),
        pltpu.VMEM((NCHT * ECH,), jnp.int32),
        pltpu.VMEM((ECH, H), jnp.float32),
        pltpu.VMEM((ECH, H), jnp.float32),
        pltpu.VMEM_SHARED((NP, H), jnp.float32),
        pltpu.SemaphoreType.DMA,
        pltpu.SemaphoreType.DMA,
    ],
)


def _tc_l1_body(x_ref, w_ref, dacc_ref, g_ref, dis_ref):
    deg = dacc_ref[0, :, 0:1] + dacc_ref[1, :, 0:1] + 1.0
    dis = lax.rsqrt(deg)
    dis_ref[...] = dis
    h = lax.dot_general(x_ref[...], w_ref[...], (((1,), (1,)), ((), ())),
                        preferred_element_type=jnp.float32)
    g_ref[...] = h * dis


def _tc_l1(x, W1, dacc):
    return pl.pallas_call(
        _tc_l1_body,
        grid=(N // BM,),
        in_specs=[
            pl.BlockSpec((BM, D), lambda i: (i, 0)),
            pl.BlockSpec((H, D), lambda i: (0, 0)),
            pl.BlockSpec((NC, BM, L), lambda i: (0, i, 0)),
        ],
        out_specs=[
            pl.BlockSpec((BM, H), lambda i: (i, 0)),
            pl.BlockSpec((BM, 1), lambda i: (i, 0)),
        ],
        out_shape=[
            jax.ShapeDtypeStruct((N, H), jnp.float32),
            jax.ShapeDtypeStruct((N, 1), jnp.float32),
        ],
    )(x, W1, dacc)


def _tc_mid_body(acc_ref, g_ref, dis_ref, b_ref, w_ref, out_ref):
    dis = dis_ref[...]
    sfull = acc_ref[0] + acc_ref[1] + g_ref[...]
    a = jnp.maximum(sfull * dis + b_ref[...], 0.0)
    out_ref[...] = lax.dot_general(a, w_ref[...], (((1,), (1,)), ((), ())),
                                   preferred_element_type=jnp.float32) * dis


def _tc_mid(acc, g, dis, b, W):
    return pl.pallas_call(
        _tc_mid_body,
        grid=(N // BM,),
        in_specs=[
            pl.BlockSpec((NC, BM, H), lambda i: (0, i, 0)),
            pl.BlockSpec((BM, H), lambda i: (i, 0)),
            pl.BlockSpec((BM, 1), lambda i: (i, 0)),
            pl.BlockSpec((1, H), lambda i: (0, 0)),
            pl.BlockSpec((H, H), lambda i: (0, 0)),
        ],
        out_specs=pl.BlockSpec((BM, H), lambda i: (i, 0)),
        out_shape=jax.ShapeDtypeStruct((N, H), jnp.float32),
    )(acc, g, dis, b, W)


def _tc_fin_body(acc_ref, g_ref, dis_ref, b_ref, wl_ref, bl_ref, out_ref):
    sfull = acc_ref[0] + acc_ref[1] + g_ref[...]
    a = jnp.maximum(sfull * dis_ref[...] + b_ref[...], 0.0)
    out_ref[...] = jnp.sum(a * wl_ref[...], axis=1, keepdims=True) + bl_ref[...]


def _tc_fin(acc, g, dis, b7, Wl, bl):
    return pl.pallas_call(
        _tc_fin_body,
        grid=(N // BM,),
        in_specs=[
            pl.BlockSpec((NC, BM, H), lambda i: (0, i, 0)),
            pl.BlockSpec((BM, H), lambda i: (i, 0)),
            pl.BlockSpec((BM, 1), lambda i: (i, 0)),
            pl.BlockSpec((1, H), lambda i: (0, 0)),
            pl.BlockSpec((1, H), lambda i: (0, 0)),
            pl.BlockSpec((1, 1), lambda i: (0, 0)),
        ],
        out_specs=pl.BlockSpec((BM, 1), lambda i: (i, 0)),
        out_shape=jax.ShapeDtypeStruct((N, 1), jnp.float32),
    )(acc, g, dis, b7, Wl, bl)


def kernel(x, edge_index, W1, b1, W2, b2, W3, b3, W4, b4, W5, b5, W6, b6,
           W7, b7, Wl, bl):
    src = edge_index[0]
    dst = edge_index[1]
    src2 = jnp.pad(src.reshape(NW, EPT), ((0, 0), (0, NCHT * ECH - EPT)),
                   constant_values=0)
    dst2 = jnp.pad(dst.reshape(NW, EPT), ((0, 0), (0, NCHT * ECH - EPT)),
                   constant_values=NP - 1)
    dacc = _sc_deg(dst2)
    g, dis = _tc_l1(x, W1, dacc)
    for Wn, bn in ((W2, b1), (W3, b2), (W4, b3), (W5, b4), (W6, b5),
                   (W7, b6)):
        acc = _sc_edge(g, src2, dst2)
        g = _tc_mid(acc, g, dis, bn.reshape(1, H), Wn)
    acc = _sc_edge(g, src2, dst2)
    out = _tc_fin(acc, g, dis, b7.reshape(1, H), Wl, bl.reshape(1, 1))
    return out.reshape(N)
